# baseline (speedup 1.0000x reference)
import functools

import jax
import jax.numpy as jnp
from jax import lax
from jax.experimental import pallas as pl
from jax.experimental.pallas import tpu as pltpu
from jax.experimental.pallas import tpu_sc as plsc

B = 16384
F = 32
L = 16
NC = 2
NS = 16
NW = NC * NS
B_PER_W = B // NW

_mesh = plsc.VectorSubcoreMesh(core_axis_name="c", subcore_axis_name="s")


@functools.partial(
    pl.kernel,
    mesh=_mesh,
    out_type=jax.ShapeDtypeStruct((B,), jnp.float32),
    compiler_params=pltpu.CompilerParams(needs_layout_passes=False,
                                         use_tc_tiling_on_sc=False),
    scratch_types=[
        pltpu.VMEM((B_PER_W,), jnp.int32),
        pltpu.VMEM((B_PER_W,), jnp.int32),
        pltpu.VMEM((B_PER_W, F), jnp.float32),
        pltpu.VMEM((B_PER_W, F), jnp.float32),
        pltpu.VMEM((B_PER_W,), jnp.float32),
        pltpu.SemaphoreType.DMA,
        pltpu.SemaphoreType.DMA,
    ],
)
def _dot_kernel(users_hbm, items_hbm, uemb_hbm, iemb_hbm, out_hbm,
                idx_u, idx_i, urows, irows, outv, sem_u, sem_i):
    wid = lax.axis_index("s") * NC + lax.axis_index("c")
    base = wid * B_PER_W

    pltpu.sync_copy(users_hbm.at[pl.ds(base, B_PER_W)], idx_u)
    pltpu.sync_copy(items_hbm.at[pl.ds(base, B_PER_W)], idx_i)

    cu = pltpu.async_copy(uemb_hbm.at[idx_u], urows, sem_u)
    ci = pltpu.async_copy(iemb_hbm.at[idx_i], irows, sem_i)
    cu.wait()
    ci.wait()

    def chunk_body(c, carry):
        rows = c * L + lax.broadcasted_iota(jnp.int32, (L,), 0)
        acc = jnp.zeros((L,), jnp.float32)
        for f in range(F):
            col = jnp.full((L,), f, jnp.int32)
            uv = plsc.load_gather(urows, [rows, col])
            iv = plsc.load_gather(irows, [rows, col])
            acc = acc + uv * iv
        outv[pl.ds(c * L, L)] = acc
        return carry

    lax.fori_loop(0, B_PER_W // L, chunk_body, 0)

    pltpu.sync_copy(outv, out_hbm.at[pl.ds(base, B_PER_W)])


def kernel(users, items, user_embeddings, item_embeddings):
    out = _dot_kernel(users.astype(jnp.int32), items.astype(jnp.int32),
                      user_embeddings, item_embeddings)
    return out.reshape(B, 1)


# speedup vs baseline: 1.3700x; 1.3700x over previous
import functools

import jax
import jax.numpy as jnp
from jax import lax
from jax.experimental import pallas as pl
from jax.experimental.pallas import tpu as pltpu
from jax.experimental.pallas import tpu_sc as plsc

B = 16384
F = 32
L = 16
NC = 2
NS = 16
NW = NC * NS
B_PER_W = B // NW
C = 128
G = 16

_mesh = plsc.VectorSubcoreMesh(core_axis_name="c", subcore_axis_name="s")


@functools.partial(
    pl.kernel,
    mesh=_mesh,
    out_type=jax.ShapeDtypeStruct((B,), jnp.float32),
    compiler_params=pltpu.CompilerParams(needs_layout_passes=False),
    scratch_types=[
        pltpu.VMEM((C,), jnp.int32),
        pltpu.VMEM((C,), jnp.int32),
        pltpu.VMEM((G, 8, F), jnp.float32),
        pltpu.VMEM((G, 8, F), jnp.float32),
        pltpu.VMEM((G, L), jnp.float32),
        pltpu.VMEM((B_PER_W,), jnp.float32),
        pltpu.SemaphoreType.DMA,
        pltpu.SemaphoreType.DMA,
    ],
)
def _dot_kernel(users_hbm, items_hbm, uemb_hbm, iemb_hbm, out_hbm,
                vidx_u, vidx_i, u_slabs, i_slabs, hbuf, outv,
                sem_u, sem_i):
    wid = lax.axis_index("s") * NC + lax.axis_index("c")
    base = wid * B_PER_W

    for k in range(B_PER_W // C):
        pltpu.sync_copy(users_hbm.at[pl.ds(base + k * C, C)], vidx_u)
        pltpu.sync_copy(items_hbm.at[pl.ds(base + k * C, C)], vidx_i)

        def group_body(g, carry):
            uvec = vidx_u[pl.ds(g * G, G)]
            ivec = vidx_i[pl.ds(g * G, G)]
            ubase = (uvec >> 3) * 8
            ibase = (ivec >> 3) * 8
            usub = uvec & 7
            isub = ivec & 7
            copies = []
            for j in range(G):
                bu = pl.multiple_of(ubase[j], 8)
                bi = pl.multiple_of(ibase[j], 8)
                copies.append(pltpu.async_copy(
                    uemb_hbm.at[pl.ds(bu, 8), :], u_slabs.at[j], sem_u))
                copies.append(pltpu.async_copy(
                    iemb_hbm.at[pl.ds(bi, 8), :], i_slabs.at[j], sem_i))
            for cp in copies:
                cp.wait()

            for j in range(G):
                su = usub[j]
                si = isub[j]
                u0 = u_slabs[j, su, pl.ds(0, L)]
                u1 = u_slabs[j, su, pl.ds(L, L)]
                i0 = i_slabs[j, si, pl.ds(0, L)]
                i1 = i_slabs[j, si, pl.ds(L, L)]
                hbuf[j, :] = u0 * i0 + u1 * i1

            rows = lax.broadcasted_iota(jnp.int32, (L,), 0)
            acc = jnp.zeros((L,), jnp.float32)
            for f in range(L):
                colf = jnp.full((L,), f, jnp.int32)
                acc = acc + plsc.load_gather(hbuf, [rows, colf])
            outv[pl.ds(carry * C + g * G, G)] = acc
            return carry

        lax.fori_loop(0, C // G, group_body, k)

    pltpu.sync_copy(outv, out_hbm.at[pl.ds(base, B_PER_W)])


def kernel(users, items, user_embeddings, item_embeddings):
    out = _dot_kernel(users.astype(jnp.int32), items.astype(jnp.int32),
                      user_embeddings, item_embeddings)
    return out.reshape(B, 1)


# speedup vs baseline: 3.4439x; 2.5139x over previous
import functools

import jax
import jax.numpy as jnp
from jax import lax
from jax.experimental import pallas as pl
from jax.experimental.pallas import tpu as pltpu
from jax.experimental.pallas import tpu_sc as plsc

B = 16384
F = 32
L = 16
NC = 2
NS = 16
NW = NC * NS
B_PER_W = B // NW
C = 128
NROWS = 1000000
RB = 65536
NBLK = 16
TOT = 4 * NBLK * 8 * RB

_mesh = plsc.VectorSubcoreMesh(core_axis_name="c", subcore_axis_name="s")


def _linearize(table_t):
    def body(x_ref, o_ref):
        for s in range(8):
            o_ref[pl.ds(s * RB, RB)] = x_ref[s, :]

    return pl.pallas_call(
        body,
        grid=(4, NBLK),
        in_specs=[pl.BlockSpec((8, RB), lambda a, b: (a, b))],
        out_specs=pl.BlockSpec((8 * RB,), lambda a, b: (a * NBLK + b,)),
        out_shape=jax.ShapeDtypeStruct((TOT,), jnp.float32),
    )(table_t)


@functools.partial(
    pl.kernel,
    mesh=_mesh,
    out_type=jax.ShapeDtypeStruct((B,), jnp.float32),
    compiler_params=pltpu.CompilerParams(needs_layout_passes=False),
    scratch_types=[
        pltpu.VMEM((C,), jnp.int32),
        pltpu.VMEM((C,), jnp.int32),
        pltpu.VMEM((F, C), jnp.int32),
        pltpu.VMEM((F, C), jnp.int32),
        pltpu.VMEM((F, C), jnp.float32),
        pltpu.VMEM((F, C), jnp.float32),
        pltpu.VMEM((B_PER_W,), jnp.float32),
        pltpu.SemaphoreType.DMA,
        pltpu.SemaphoreType.DMA,
    ],
)
def _dot_kernel(users_hbm, items_hbm, ue1d, ie1d, out_hbm,
                vidx_u, vidx_i, idxu2, idxi2, ut, it, outv, sem_u, sem_i):
    wid = lax.axis_index("s") * NC + lax.axis_index("c")
    base = wid * B_PER_W

    def chunk_body(k, carry):
        pltpu.sync_copy(users_hbm.at[pl.ds(base + k * C, C)], vidx_u)
        pltpu.sync_copy(items_hbm.at[pl.ds(base + k * C, C)], vidx_i)

        for h in range(C // L):
            uvec = vidx_u[pl.ds(h * L, L)]
            ivec = vidx_i[pl.ds(h * L, L)]
            ua = ((uvec >> 16) << 19) + (uvec & 0xFFFF)
            ia = ((ivec >> 16) << 19) + (ivec & 0xFFFF)
            for f in range(F):
                af = (f >> 3) * 8388608 + (f & 7) * 65536
                idxu2[f, pl.ds(h * L, L)] = ua + af
                idxi2[f, pl.ds(h * L, L)] = ia + af

        copies = []
        for f in range(F):
            copies.append(pltpu.async_copy(
                ue1d.at[idxu2.at[f]], ut.at[f], sem_u))
            copies.append(pltpu.async_copy(
                ie1d.at[idxi2.at[f]], it.at[f], sem_i))
        for cp in copies:
            cp.wait()

        for g in range(C // L):
            acc = jnp.zeros((L,), jnp.float32)
            for f in range(F):
                acc = acc + ut[f, pl.ds(g * L, L)] * it[f, pl.ds(g * L, L)]
            outv[pl.ds(k * C + g * L, L)] = acc
        return carry

    lax.fori_loop(0, B_PER_W // C, chunk_body, 0)

    pltpu.sync_copy(outv, out_hbm.at[pl.ds(base, B_PER_W)])


def kernel(users, items, user_embeddings, item_embeddings):
    ue1d = _linearize(jnp.swapaxes(user_embeddings, 0, 1))
    ie1d = _linearize(jnp.swapaxes(item_embeddings, 0, 1))
    out = _dot_kernel(users.astype(jnp.int32), items.astype(jnp.int32),
                      ue1d, ie1d)
    return out.reshape(B, 1)
